# baseline (device time: 2572920 ns/iter reference)
import jax
import jax.numpy as jnp
from jax import lax
from jax.experimental import pallas as pl
from jax.experimental.pallas import tpu as pltpu


def kernel(ids, E):
    T = ids.shape[0]
    V_local, D = E.shape

    z = lax.axis_index("z")
    local = ids - z * V_local
    mask = (local >= 0) & (local < V_local)
    safe = jnp.where(mask, local, 0)
    partial = jnp.where(mask[:, None], jnp.take(E, safe, axis=0), 0.0)

    CHUNK = 512
    NC = T // CHUNK

    def body(partial_ref, out_ref, l_buf, r_buf,
             load_sem, store_sem, send_sem, recv_sem, credit_sem):
        my_x = lax.axis_index("x")
        my_y = lax.axis_index("y")
        my_z = lax.axis_index("z")
        peer = (my_x, my_y, 1 - my_z)

        barrier = pltpu.get_barrier_semaphore()
        pl.semaphore_signal(barrier, inc=1, device_id=peer,
                            device_id_type=pl.DeviceIdType.MESH)
        pl.semaphore_wait(barrier, 1)

        for c in range(NC):
            rows = pl.ds(c * CHUNK, CHUNK)
            load = pltpu.make_async_copy(partial_ref.at[rows, :], l_buf, load_sem)
            load.start()
            load.wait()
            if c > 0:
                pl.semaphore_wait(credit_sem, 1)
            rdma = pltpu.make_async_remote_copy(
                src_ref=l_buf, dst_ref=r_buf,
                send_sem=send_sem, recv_sem=recv_sem,
                device_id=peer, device_id_type=pl.DeviceIdType.MESH)
            rdma.start()
            rdma.wait()
            r_buf[...] = l_buf[...] + r_buf[...]
            store = pltpu.make_async_copy(r_buf, out_ref.at[rows, :], store_sem)
            store.start()
            store.wait()
            if c < NC - 1:
                pl.semaphore_signal(credit_sem, inc=1, device_id=peer,
                                    device_id_type=pl.DeviceIdType.MESH)

    out = pl.pallas_call(
        body,
        out_shape=jax.ShapeDtypeStruct((T, D), jnp.float32),
        in_specs=[pl.BlockSpec(memory_space=pl.ANY)],
        out_specs=pl.BlockSpec(memory_space=pl.ANY),
        scratch_shapes=[
            pltpu.VMEM((CHUNK, D), jnp.float32),
            pltpu.VMEM((CHUNK, D), jnp.float32),
            pltpu.SemaphoreType.DMA,
            pltpu.SemaphoreType.DMA,
            pltpu.SemaphoreType.DMA,
            pltpu.SemaphoreType.DMA,
            pltpu.SemaphoreType.REGULAR,
        ],
        compiler_params=pltpu.CompilerParams(collective_id=0),
    )(partial)
    return out


# device time: 516336 ns/iter; 4.9830x vs baseline; 4.9830x over previous
import jax
import jax.numpy as jnp
from jax import lax
from jax.experimental import pallas as pl
from jax.experimental.pallas import tpu as pltpu

VB = 512
CHUNK = 256


def kernel(ids, E):
    T = ids.shape[0]
    V_local, D = E.shape
    HALF = T // 2
    NB = V_local // VB
    NC = HALF // CHUNK

    ids2 = ids.reshape(T, 1).astype(jnp.int32)

    def body(ids_ref, e_ref, out_ref,
             e_stage, acc, z_rbuf, red, x_rbuf,
             e_sem, st_sem,
             z_send, z_recv, x_send, x_recv,
             z_credit, x_credit):
        my_x = lax.axis_index("x")
        my_y = lax.axis_index("y")
        my_z = lax.axis_index("z")
        peer_z = (my_x, my_y, 1 - my_z)
        peer_x = (1 - my_x, my_y, my_z)

        barrier = pltpu.get_barrier_semaphore()
        for nbr in (peer_z, peer_x):
            pl.semaphore_signal(barrier, inc=1, device_id=nbr,
                                device_id_type=pl.DeviceIdType.MESH)
        pl.semaphore_wait(barrier, 2)

        my_base = my_x * HALF
        ids_my = ids_ref[pl.ds(my_base, HALF), :]
        voff = my_z * V_local
        col = lax.broadcasted_iota(jnp.int32, (1, VB), 1)

        def e_load(b, slot):
            return pltpu.make_async_copy(
                e_ref.at[pl.ds(b * VB, VB), :], e_stage.at[slot], e_sem.at[slot])

        e_load(0, 0).start()
        for b in range(NB):
            slot = b % 2
            e_load(b, slot).wait()
            if b + 1 < NB:
                e_load(b + 1, (b + 1) % 2).start()
            oh = (ids_my == (voff + b * VB + col)).astype(jnp.float32)
            mm = jax.lax.dot_general(
                oh, e_stage[slot],
                (((1,), (0,)), ((), ())),
                preferred_element_type=jnp.float32)
            if b == 0:
                acc[...] = mm
            else:
                acc[...] = acc[...] + mm

        def z_desc(c):
            slot = c % 2
            return pltpu.make_async_remote_copy(
                src_ref=acc.at[pl.ds(c * CHUNK, CHUNK), :],
                dst_ref=z_rbuf.at[slot],
                send_sem=z_send.at[slot], recv_sem=z_recv.at[slot],
                device_id=peer_z, device_id_type=pl.DeviceIdType.MESH)

        def x_desc(c):
            slot = c % 2
            return pltpu.make_async_remote_copy(
                src_ref=red.at[slot],
                dst_ref=x_rbuf.at[slot],
                send_sem=x_send.at[slot], recv_sem=x_recv.at[slot],
                device_id=peer_x, device_id_type=pl.DeviceIdType.MESH)

        for c in range(NC):
            slot = c % 2
            if c > 0:
                pl.semaphore_wait(z_credit, 1)
            zd = z_desc(c)
            zd.start()
            zd.wait()
            red[slot] = acc[pl.ds(c * CHUNK, CHUNK), :] + z_rbuf[slot]
            if c < NC - 1:
                pl.semaphore_signal(z_credit, inc=1, device_id=peer_z,
                                    device_id_type=pl.DeviceIdType.MESH)
            st = pltpu.make_async_copy(
                red.at[slot], out_ref.at[pl.ds(my_base + c * CHUNK, CHUNK), :],
                st_sem)
            st.start()
            st.wait()
            if c > 0:
                pl.semaphore_wait(x_credit, 1)
            xd = x_desc(c)
            xd.start()
            xd.wait()
            st2 = pltpu.make_async_copy(
                x_rbuf.at[slot],
                out_ref.at[pl.ds((1 - my_x) * HALF + c * CHUNK, CHUNK), :],
                st_sem)
            st2.start()
            st2.wait()
            if c < NC - 1:
                pl.semaphore_signal(x_credit, inc=1, device_id=peer_x,
                                    device_id_type=pl.DeviceIdType.MESH)

    out = pl.pallas_call(
        body,
        out_shape=jax.ShapeDtypeStruct((T, D), jnp.float32),
        in_specs=[
            pl.BlockSpec(memory_space=pltpu.MemorySpace.VMEM),
            pl.BlockSpec(memory_space=pl.ANY),
        ],
        out_specs=pl.BlockSpec(memory_space=pl.ANY),
        scratch_shapes=[
            pltpu.VMEM((2, VB, D), jnp.float32),
            pltpu.VMEM((HALF, D), jnp.float32),
            pltpu.VMEM((2, CHUNK, D), jnp.float32),
            pltpu.VMEM((2, CHUNK, D), jnp.float32),
            pltpu.VMEM((2, CHUNK, D), jnp.float32),
            pltpu.SemaphoreType.DMA((2,)),
            pltpu.SemaphoreType.DMA,
            pltpu.SemaphoreType.DMA((2,)),
            pltpu.SemaphoreType.DMA((2,)),
            pltpu.SemaphoreType.DMA((2,)),
            pltpu.SemaphoreType.DMA((2,)),
            pltpu.SemaphoreType.REGULAR,
            pltpu.SemaphoreType.REGULAR,
        ],
        compiler_params=pltpu.CompilerParams(
            collective_id=0, vmem_limit_bytes=100 * 1024 * 1024),
    )(ids2, E)
    return out


# device time: 314114 ns/iter; 8.1910x vs baseline; 1.6438x over previous
import jax
import jax.numpy as jnp
from jax import lax
from jax.experimental import pallas as pl
from jax.experimental.pallas import tpu as pltpu

VB = 512
CHUNK = 256


def kernel(ids, E):
    T = ids.shape[0]
    V_local, D = E.shape
    HALF = T // 2
    NB = V_local // VB
    NC = HALF // CHUNK

    ids2 = ids.reshape(T, 1).astype(jnp.int32)

    def body(ids_ref, e_ref, out_ref,
             e_stage, acc, z_rbuf, red, x_rbuf,
             e_sem, st_sem,
             z_send, z_recv, x_send, x_recv,
             z_credit, x_credit):
        my_x = lax.axis_index("x")
        my_y = lax.axis_index("y")
        my_z = lax.axis_index("z")
        peer_z = (my_x, my_y, 1 - my_z)
        peer_x = (1 - my_x, my_y, my_z)

        barrier = pltpu.get_barrier_semaphore()
        for nbr in (peer_z, peer_x):
            pl.semaphore_signal(barrier, inc=1, device_id=nbr,
                                device_id_type=pl.DeviceIdType.MESH)
        pl.semaphore_wait(barrier, 2)

        my_base = my_x * HALF
        ids_my = ids_ref[pl.ds(my_base, HALF), :]
        voff = my_z * V_local
        col = lax.broadcasted_iota(jnp.int32, (1, VB), 1)

        def e_load(b, slot):
            return pltpu.make_async_copy(
                e_ref.at[pl.ds(b * VB, VB), :], e_stage.at[slot], e_sem.at[slot])

        e_load(0, 0).start()
        for b in range(NB):
            slot = b % 2
            e_load(b, slot).wait()
            if b + 1 < NB:
                e_load(b + 1, (b + 1) % 2).start()
            oh = (ids_my == (voff + b * VB + col)).astype(jnp.float32)
            mm = jax.lax.dot_general(
                oh, e_stage[slot],
                (((1,), (0,)), ((), ())),
                preferred_element_type=jnp.float32)
            if b == 0:
                acc[...] = mm
            else:
                acc[...] = acc[...] + mm

        def z_desc(c):
            slot = c % 2
            return pltpu.make_async_remote_copy(
                src_ref=acc.at[pl.ds(c * CHUNK, CHUNK), :],
                dst_ref=z_rbuf.at[slot],
                send_sem=z_send.at[slot], recv_sem=z_recv.at[slot],
                device_id=peer_z, device_id_type=pl.DeviceIdType.MESH)

        def x_desc(c):
            slot = c % 2
            return pltpu.make_async_remote_copy(
                src_ref=red.at[slot],
                dst_ref=x_rbuf.at[slot],
                send_sem=x_send.at[slot], recv_sem=x_recv.at[slot],
                device_id=peer_x, device_id_type=pl.DeviceIdType.MESH)

        z_desc(0).start()
        for c in range(NC):
            slot = c % 2
            if c + 1 < NC:
                if c + 1 >= 2:
                    z_desc(c + 1).wait_send()
                    pl.semaphore_wait(z_credit, 1)
                z_desc(c + 1).start()
            z_desc(c).wait_recv()
            if c >= 2:
                x_desc(c).wait_send()
            red[slot] = acc[pl.ds(c * CHUNK, CHUNK), :] + z_rbuf[slot]
            if c < NC - 2:
                pl.semaphore_signal(z_credit, inc=1, device_id=peer_z,
                                    device_id_type=pl.DeviceIdType.MESH)
            if c >= 2:
                pl.semaphore_wait(x_credit, 1)
            x_desc(c).start()
            st = pltpu.make_async_copy(
                red.at[slot], out_ref.at[pl.ds(my_base + c * CHUNK, CHUNK), :],
                st_sem)
            st.start()
            st.wait()
            if c >= 1:
                x_desc(c - 1).wait_recv()
                st2 = pltpu.make_async_copy(
                    x_rbuf.at[(c - 1) % 2],
                    out_ref.at[
                        pl.ds((1 - my_x) * HALF + (c - 1) * CHUNK, CHUNK), :],
                    st_sem)
                st2.start()
                st2.wait()
                if c - 1 < NC - 2:
                    pl.semaphore_signal(x_credit, inc=1, device_id=peer_x,
                                        device_id_type=pl.DeviceIdType.MESH)

        x_desc(NC - 1).wait_recv()
        st2 = pltpu.make_async_copy(
            x_rbuf.at[(NC - 1) % 2],
            out_ref.at[pl.ds((1 - my_x) * HALF + (NC - 1) * CHUNK, CHUNK), :],
            st_sem)
        st2.start()
        st2.wait()
        z_desc(NC - 2).wait_send()
        z_desc(NC - 1).wait_send()
        x_desc(NC - 2).wait_send()
        x_desc(NC - 1).wait_send()

    out = pl.pallas_call(
        body,
        out_shape=jax.ShapeDtypeStruct((T, D), jnp.float32),
        in_specs=[
            pl.BlockSpec(memory_space=pltpu.MemorySpace.VMEM),
            pl.BlockSpec(memory_space=pl.ANY),
        ],
        out_specs=pl.BlockSpec(memory_space=pl.ANY),
        scratch_shapes=[
            pltpu.VMEM((2, VB, D), jnp.float32),
            pltpu.VMEM((HALF, D), jnp.float32),
            pltpu.VMEM((2, CHUNK, D), jnp.float32),
            pltpu.VMEM((2, CHUNK, D), jnp.float32),
            pltpu.VMEM((2, CHUNK, D), jnp.float32),
            pltpu.SemaphoreType.DMA((2,)),
            pltpu.SemaphoreType.DMA,
            pltpu.SemaphoreType.DMA((2,)),
            pltpu.SemaphoreType.DMA((2,)),
            pltpu.SemaphoreType.DMA((2,)),
            pltpu.SemaphoreType.DMA((2,)),
            pltpu.SemaphoreType.REGULAR,
            pltpu.SemaphoreType.REGULAR,
        ],
        compiler_params=pltpu.CompilerParams(
            collective_id=0, vmem_limit_bytes=100 * 1024 * 1024),
    )(ids2, E)
    return out
